# Initial kernel scaffold; baseline (speedup 1.0000x reference)
import jax
import jax.numpy as jnp
from jax.experimental import pallas as pl


def kernel(x, edge_index, W1, W2):
    raise NotImplementedError("write your pallas kernel here")


# speedup vs baseline: 4.5109x; 4.5109x over previous
import functools

import jax
import jax.numpy as jnp
from jax import lax
from jax.experimental import pallas as pl
from jax.experimental.pallas import tpu as pltpu
from jax.experimental.pallas import tpu_sc as plsc

N_NODES = 10000
N_EDGES = 160000
D = 256
DQ = 64

NC = 2
NS = 16
CHUNK = 128
NCHUNKS = N_EDGES // CHUNK
CPT = 79
CPT_LAST = NCHUNKS - 15 * CPT
N_PAD = 10240
STRIPE = N_PAD // NS
NDEG = 640
DSTRIPE = NDEG // NS


def _sc_aggregate(xq0, xq1, xq2, xq3, src_c, dst_c):
    mesh = plsc.VectorSubcoreMesh(core_axis_name="c", subcore_axis_name="s")

    @functools.partial(
        pl.kernel,
        out_type=(
            jax.ShapeDtypeStruct((N_PAD, DQ), jnp.float32),
            jax.ShapeDtypeStruct((N_PAD, DQ), jnp.float32),
            jax.ShapeDtypeStruct((N_PAD, DQ), jnp.float32),
            jax.ShapeDtypeStruct((N_PAD, DQ), jnp.float32),
            jax.ShapeDtypeStruct((NDEG, 16), jnp.float32),
            jax.ShapeDtypeStruct((NDEG, 16), jnp.float32),
        ),
        mesh=mesh,
        compiler_params=pltpu.CompilerParams(use_tc_tiling_on_sc=False),
        scratch_types=[
            pltpu.VMEM((CPT, CHUNK), jnp.int32),
            pltpu.VMEM((CPT, CHUNK), jnp.int32),
            pltpu.VMEM((CHUNK, DQ), jnp.float32),
            pltpu.VMEM((CHUNK, 16), jnp.float32),
            pltpu.VMEM((CHUNK,), jnp.int32),
            pltpu.VMEM((32, DQ), jnp.float32),
            pltpu.VMEM((DSTRIPE, 16), jnp.float32),
            pltpu.VMEM_SHARED((N_PAD, DQ), jnp.float32),
            pltpu.VMEM_SHARED((NDEG, 16), jnp.float32),
            pltpu.SemaphoreType.DMA,
        ],
    )
    def k(xq0_hbm, xq1_hbm, xq2_hbm, xq3_hbm, src_hbm, dst_hbm,
          a0_hbm, a1_hbm, a2_hbm, a3_hbm, deg0_hbm, deg1_hbm,
          src_v, dst_v, rows_v, onehot_v, rowidx_v, zb_v, zd_v,
          agg_sh, deg_sh, sem):
        c = lax.axis_index("c")
        s = lax.axis_index("s")
        cnt = jnp.where(s == NS - 1, CPT_LAST, CPT)

        zeros16 = jnp.zeros((16,), jnp.float32)
        ones16 = jnp.full((16,), 1.0, jnp.float32)
        iota16 = lax.iota(jnp.int32, 16)

        def init_zb(i, _):
            for kk in range(DQ // 16):
                zb_v[i, pl.ds(kk * 16, 16)] = zeros16
            return 0
        lax.fori_loop(0, 32, init_zb, 0)

        def init_zd(i, _):
            zd_v[i, :] = zeros16
            return 0
        lax.fori_loop(0, DSTRIPE, init_zd, 0)

        def zero_agg_stripe():
            def zero_one(r, _):
                pltpu.sync_copy(zb_v,
                                agg_sh.at[pl.ds(s * STRIPE + r * 32, 32)])
                return 0
            lax.fori_loop(0, STRIPE // 32, zero_one, 0)

        zero_agg_stripe()
        pltpu.sync_copy(zd_v, deg_sh.at[pl.ds(s * DSTRIPE, DSTRIPE)])

        pltpu.sync_copy(src_hbm.at[s], src_v)
        pltpu.sync_copy(dst_hbm.at[s], dst_v)

        plsc.subcore_barrier()

        def deg_work(j):
            for g in range(CHUNK // 16):
                d16 = dst_v[j, pl.ds(g * 16, 16)]
                col = jnp.bitwise_and(d16, 15)
                rowidx_v[pl.ds(g * 16, 16)] = jnp.right_shift(d16, 4)
                for r in range(16):
                    onehot_v[g * 16 + r, :] = jnp.where(
                        iota16 == col[r], ones16, zeros16)
            pltpu.sync_copy(onehot_v, deg_sh.at[rowidx_v], add=True)

        def edge_loop(xh_hbm, with_deg):
            def body(j, _):
                pltpu.async_copy(xh_hbm.at[src_v.at[j]], rows_v, sem).wait()
                pltpu.sync_copy(rows_v, agg_sh.at[dst_v.at[j]], add=True)
                if with_deg:
                    pl.when(jnp.bitwise_and(j, 1) == c)(lambda: deg_work(j))
                return 0
            lax.fori_loop(0, cnt, body, 0)

        def copy_agg_out(aq_hbm):
            pltpu.sync_copy(agg_sh.at[pl.ds(s * STRIPE, STRIPE)],
                            aq_hbm.at[pl.ds(s * STRIPE, STRIPE)])

        def run_core(xa_hbm, xb_hbm, aa_hbm, ab_hbm, deg_hbm):
            edge_loop(xa_hbm, True)
            plsc.subcore_barrier()
            copy_agg_out(aa_hbm)
            pltpu.sync_copy(deg_sh.at[pl.ds(s * DSTRIPE, DSTRIPE)],
                            deg_hbm.at[pl.ds(s * DSTRIPE, DSTRIPE)])
            zero_agg_stripe()
            plsc.subcore_barrier()
            edge_loop(xb_hbm, False)
            plsc.subcore_barrier()
            copy_agg_out(ab_hbm)

        pl.when(c == 0)(
            lambda: run_core(xq0_hbm, xq1_hbm, a0_hbm, a1_hbm, deg0_hbm))
        pl.when(c == 1)(
            lambda: run_core(xq2_hbm, xq3_hbm, a2_hbm, a3_hbm, deg1_hbm))

    return k(xq0, xq1, xq2, xq3, src_c, dst_c)


def _tc_body(a0_ref, a1_ref, a2_ref, a3_ref, d0_ref, d1_ref, x_ref,
             w0_ref, w1_ref, w2_ref, w3_ref, ws_ref, o_ref):
    deg = jnp.maximum(d0_ref[:] + d1_ref[:], 1.0)
    acc = jnp.dot(x_ref[:], ws_ref[:], preferred_element_type=jnp.float32)
    acc += jnp.dot(a0_ref[:] / deg, w0_ref[:],
                   preferred_element_type=jnp.float32)
    acc += jnp.dot(a1_ref[:] / deg, w1_ref[:],
                   preferred_element_type=jnp.float32)
    acc += jnp.dot(a2_ref[:] / deg, w2_ref[:],
                   preferred_element_type=jnp.float32)
    acc += jnp.dot(a3_ref[:] / deg, w3_ref[:],
                   preferred_element_type=jnp.float32)
    o_ref[:] = acc


def _tc_epilogue(aggs, deg0_col, deg1_col, x, w1q_t, w2_t):
    blk = 1000
    grid = (N_NODES // blk,)
    return pl.pallas_call(
        _tc_body,
        grid=grid,
        in_specs=[
            pl.BlockSpec((blk, DQ), lambda i: (i, 0)),
            pl.BlockSpec((blk, DQ), lambda i: (i, 0)),
            pl.BlockSpec((blk, DQ), lambda i: (i, 0)),
            pl.BlockSpec((blk, DQ), lambda i: (i, 0)),
            pl.BlockSpec((blk, 1), lambda i: (i, 0)),
            pl.BlockSpec((blk, 1), lambda i: (i, 0)),
            pl.BlockSpec((blk, D), lambda i: (i, 0)),
            pl.BlockSpec((DQ, D), lambda i: (0, 0)),
            pl.BlockSpec((DQ, D), lambda i: (0, 0)),
            pl.BlockSpec((DQ, D), lambda i: (0, 0)),
            pl.BlockSpec((DQ, D), lambda i: (0, 0)),
            pl.BlockSpec((D, D), lambda i: (0, 0)),
        ],
        out_specs=pl.BlockSpec((blk, D), lambda i: (i, 0)),
        out_shape=jax.ShapeDtypeStruct((N_NODES, D), jnp.float32),
    )(*aggs, deg0_col, deg1_col, x, *w1q_t, w2_t)


@jax.jit
def kernel(x, edge_index, W1, W2):
    pad = NS * CPT * CHUNK - N_EDGES
    src_c = jnp.concatenate(
        [edge_index[0].astype(jnp.int32), jnp.zeros((pad,), jnp.int32)]
    ).reshape(NS, CPT, CHUNK)
    dst_c = jnp.concatenate(
        [edge_index[1].astype(jnp.int32), jnp.zeros((pad,), jnp.int32)]
    ).reshape(NS, CPT, CHUNK)
    xq = [x[:, i * DQ:(i + 1) * DQ] for i in range(4)]
    a0, a1, a2, a3, deg0, deg1 = _sc_aggregate(*xq, src_c, dst_c)
    deg0_col = deg0.reshape(-1)[:N_NODES].reshape(N_NODES, 1)
    deg1_col = deg1.reshape(-1)[:N_NODES].reshape(N_NODES, 1)
    w1q_t = [W1[:, i * DQ:(i + 1) * DQ].T for i in range(4)]
    w2_t = W2.T
    return _tc_epilogue([a0, a1, a2, a3], deg0_col, deg1_col, x, w1q_t, w2_t)


# speedup vs baseline: 6.5826x; 1.4593x over previous
import functools

import jax
import jax.numpy as jnp
from jax import lax
from jax.experimental import pallas as pl
from jax.experimental.pallas import tpu as pltpu
from jax.experimental.pallas import tpu_sc as plsc

N_NODES = 10000
N_EDGES = 160000
D = 256
DQ = 64

NC = 2
NS = 16
CHUNK = 128
NCHUNKS = N_EDGES // CHUNK
CPT = 79
CPT_LAST = NCHUNKS - 15 * CPT
N_PAD = 10240
STRIPE = N_PAD // NS
NDEG = 640
DSTRIPE = NDEG // NS


def _sc_aggregate(xf, src4_c, dst_c):
    mesh = plsc.VectorSubcoreMesh(core_axis_name="c", subcore_axis_name="s")

    @functools.partial(
        pl.kernel,
        out_type=(
            jax.ShapeDtypeStruct((N_PAD, DQ), jnp.float32),
            jax.ShapeDtypeStruct((N_PAD, DQ), jnp.float32),
            jax.ShapeDtypeStruct((N_PAD, DQ), jnp.float32),
            jax.ShapeDtypeStruct((N_PAD, DQ), jnp.float32),
            jax.ShapeDtypeStruct((NDEG, 16), jnp.float32),
            jax.ShapeDtypeStruct((NDEG, 16), jnp.float32),
        ),
        mesh=mesh,
        compiler_params=pltpu.CompilerParams(use_tc_tiling_on_sc=False),
        scratch_types=[
            pltpu.VMEM((CPT, CHUNK), jnp.int32),
            pltpu.VMEM((CPT, CHUNK), jnp.int32),
            pltpu.VMEM((CPT, CHUNK), jnp.int32),
            pltpu.VMEM((CHUNK, DQ), jnp.float32),
            pltpu.VMEM((CHUNK, DQ), jnp.float32),
            pltpu.VMEM((CHUNK, 16), jnp.float32),
            pltpu.VMEM((CHUNK,), jnp.int32),
            pltpu.VMEM((32, DQ), jnp.float32),
            pltpu.VMEM((DSTRIPE, 16), jnp.float32),
            pltpu.VMEM_SHARED((N_PAD, DQ), jnp.float32),
            pltpu.VMEM_SHARED((NDEG, 16), jnp.float32),
            pltpu.SemaphoreType.DMA,
            pltpu.SemaphoreType.DMA,
            pltpu.SemaphoreType.DMA,
            pltpu.SemaphoreType.DMA,
            pltpu.SemaphoreType.DMA,
        ],
    )
    def k(xf_hbm, src4_hbm, dst_hbm,
          a0_hbm, a1_hbm, a2_hbm, a3_hbm, deg0_hbm, deg1_hbm,
          src4_v, dst_v, idx_v, rows_a, rows_b, onehot_v, rowidx_v,
          zb_v, zd_v, agg_sh, deg_sh, gsem0, gsem1, ssem0, ssem1, dsem):
        c = lax.axis_index("c")
        s = lax.axis_index("s")
        cnt = jnp.where(s == NS - 1, CPT_LAST, CPT)
        npairs = (cnt - 1) // 2

        zeros16 = jnp.zeros((16,), jnp.float32)
        ones16 = jnp.full((16,), 1.0, jnp.float32)
        iota16 = lax.iota(jnp.int32, 16)

        def init_zb(i, _):
            for kk in range(DQ // 16):
                zb_v[i, pl.ds(kk * 16, 16)] = zeros16
            return 0
        lax.fori_loop(0, 32, init_zb, 0)

        def init_zd(i, _):
            zd_v[i, :] = zeros16
            return 0
        lax.fori_loop(0, DSTRIPE, init_zd, 0)

        def zero_agg_stripe():
            def zero_one(r, _):
                pltpu.sync_copy(zb_v,
                                agg_sh.at[pl.ds(s * STRIPE + r * 32, 32)])
                return 0
            lax.fori_loop(0, STRIPE // 32, zero_one, 0)

        zero_agg_stripe()
        pltpu.sync_copy(zd_v, deg_sh.at[pl.ds(s * DSTRIPE, DSTRIPE)])

        pltpu.sync_copy(src4_hbm.at[s], src4_v)
        pltpu.sync_copy(dst_hbm.at[s], dst_v)

        def start_gather(j, buf, sem):
            pltpu.async_copy(xf_hbm.at[idx_v.at[j]], buf, sem)

        def wait_gather(j, buf, sem):
            pltpu.make_async_copy(xf_hbm.at[idx_v.at[j]], buf, sem).wait()

        def start_scat(j, buf, sem):
            pltpu.async_copy(buf, agg_sh.at[dst_v.at[j]], sem, add=True)

        def wait_scat(j, buf, sem):
            pltpu.make_async_copy(buf, agg_sh.at[dst_v.at[j]], sem).wait()

        def build_onehot(j):
            for g in range(CHUNK // 16):
                d16 = dst_v[j, pl.ds(g * 16, 16)]
                col = jnp.bitwise_and(d16, 15)
                rowidx_v[pl.ds(g * 16, 16)] = jnp.right_shift(d16, 4)
                for r in range(16):
                    onehot_v[g * 16 + r, :] = jnp.where(
                        iota16 == col[r], ones16, zeros16)

        def wait_deg():
            pltpu.make_async_copy(onehot_v, deg_sh.at[rowidx_v], dsem).wait()

        def run_pass(q, with_deg):
            def bld(r, _):
                for kk in range(CHUNK // 16):
                    sl = pl.ds(kk * 16, 16)
                    idx_v[r, sl] = src4_v[r, sl] + q
                return 0
            lax.fori_loop(0, CPT, bld, 0)

            start_gather(0, rows_a, gsem0)
            start_gather(1, rows_b, gsem1)
            plsc.subcore_barrier()

            def pair(p, _):
                j0 = 2 * p
                j1 = j0 + 1
                wait_gather(j0, rows_a, gsem0)
                start_scat(j0, rows_a, ssem0)
                if with_deg:
                    jd = j0 + c
                    pl.when(p > 0)(wait_deg)
                    build_onehot(jd)
                    pltpu.async_copy(onehot_v, deg_sh.at[rowidx_v], dsem,
                                     add=True)
                wait_gather(j1, rows_b, gsem1)
                start_scat(j1, rows_b, ssem1)
                wait_scat(j0, rows_a, ssem0)
                pl.when(j0 + 2 < cnt)(
                    lambda: start_gather(j0 + 2, rows_a, gsem0))
                wait_scat(j1, rows_b, ssem1)
                pl.when(j1 + 2 < cnt)(
                    lambda: start_gather(j1 + 2, rows_b, gsem1))
                return 0
            lax.fori_loop(0, npairs, pair, 0)

            if with_deg:
                wait_deg()

            jl = cnt - 1
            wait_gather(jl, rows_a, gsem0)
            pltpu.sync_copy(rows_a, agg_sh.at[dst_v.at[jl]], add=True)
            if with_deg:
                def tail_deg():
                    build_onehot(jl)
                    pltpu.sync_copy(onehot_v, deg_sh.at[rowidx_v], add=True)
                pl.when(c == 0)(tail_deg)

            plsc.subcore_barrier()

        def copy_agg_out(aq_hbm):
            pltpu.sync_copy(agg_sh.at[pl.ds(s * STRIPE, STRIPE)],
                            aq_hbm.at[pl.ds(s * STRIPE, STRIPE)])

        def copy_deg_out(deg_hbm):
            pltpu.sync_copy(deg_sh.at[pl.ds(s * DSTRIPE, DSTRIPE)],
                            deg_hbm.at[pl.ds(s * DSTRIPE, DSTRIPE)])

        run_pass(2 * c, True)

        def out_c0():
            copy_agg_out(a0_hbm)
            copy_deg_out(deg0_hbm)

        def out_c1():
            copy_agg_out(a2_hbm)
            copy_deg_out(deg1_hbm)

        pl.when(c == 0)(out_c0)
        pl.when(c == 1)(out_c1)
        zero_agg_stripe()
        plsc.subcore_barrier()

        run_pass(2 * c + 1, False)
        pl.when(c == 0)(lambda: copy_agg_out(a1_hbm))
        pl.when(c == 1)(lambda: copy_agg_out(a3_hbm))

    return k(xf, src4_c, dst_c)


def _tc_body(a0_ref, a1_ref, a2_ref, a3_ref, d0_ref, d1_ref, x_ref,
             w0_ref, w1_ref, w2_ref, w3_ref, ws_ref, o_ref):
    deg = jnp.maximum(d0_ref[:] + d1_ref[:], 1.0)
    acc = jnp.dot(x_ref[:], ws_ref[:], preferred_element_type=jnp.float32)
    acc += jnp.dot(a0_ref[:] / deg, w0_ref[:],
                   preferred_element_type=jnp.float32)
    acc += jnp.dot(a1_ref[:] / deg, w1_ref[:],
                   preferred_element_type=jnp.float32)
    acc += jnp.dot(a2_ref[:] / deg, w2_ref[:],
                   preferred_element_type=jnp.float32)
    acc += jnp.dot(a3_ref[:] / deg, w3_ref[:],
                   preferred_element_type=jnp.float32)
    o_ref[:] = acc


def _tc_epilogue(aggs, deg0_col, deg1_col, x, w1q_t, w2_t):
    blk = 1000
    grid = (N_NODES // blk,)
    return pl.pallas_call(
        _tc_body,
        grid=grid,
        in_specs=[
            pl.BlockSpec((blk, DQ), lambda i: (i, 0)),
            pl.BlockSpec((blk, DQ), lambda i: (i, 0)),
            pl.BlockSpec((blk, DQ), lambda i: (i, 0)),
            pl.BlockSpec((blk, DQ), lambda i: (i, 0)),
            pl.BlockSpec((blk, 1), lambda i: (i, 0)),
            pl.BlockSpec((blk, 1), lambda i: (i, 0)),
            pl.BlockSpec((blk, D), lambda i: (i, 0)),
            pl.BlockSpec((DQ, D), lambda i: (0, 0)),
            pl.BlockSpec((DQ, D), lambda i: (0, 0)),
            pl.BlockSpec((DQ, D), lambda i: (0, 0)),
            pl.BlockSpec((DQ, D), lambda i: (0, 0)),
            pl.BlockSpec((D, D), lambda i: (0, 0)),
        ],
        out_specs=pl.BlockSpec((blk, D), lambda i: (i, 0)),
        out_shape=jax.ShapeDtypeStruct((N_NODES, D), jnp.float32),
    )(*aggs, deg0_col, deg1_col, x, *w1q_t, w2_t)


@jax.jit
def kernel(x, edge_index, W1, W2):
    pad = NS * CPT * CHUNK - N_EDGES
    src4_c = jnp.concatenate(
        [edge_index[0].astype(jnp.int32) * 4, jnp.zeros((pad,), jnp.int32)]
    ).reshape(NS, CPT, CHUNK)
    dst_c = jnp.concatenate(
        [edge_index[1].astype(jnp.int32), jnp.zeros((pad,), jnp.int32)]
    ).reshape(NS, CPT, CHUNK)
    xf = x.reshape(N_NODES * 4, DQ)
    a0, a1, a2, a3, deg0, deg1 = _sc_aggregate(xf, src4_c, dst_c)
    deg0_col = deg0.reshape(-1)[:N_NODES].reshape(N_NODES, 1)
    deg1_col = deg1.reshape(-1)[:N_NODES].reshape(N_NODES, 1)
    w1q_t = [W1[:, i * DQ:(i + 1) * DQ].T for i in range(4)]
    w2_t = W2.T
    return _tc_epilogue([a0, a1, a2, a3], deg0_col, deg1_col, x, w1q_t, w2_t)
